# baseline (speedup 1.0000x reference)
import functools

import jax
import jax.numpy as jnp
from jax import lax
from jax.experimental import pallas as pl
from jax.experimental.pallas import tpu as pltpu
from jax.experimental.pallas import tpu_sc as plsc

H, W = 224, 448
RG = 16
RH = H // RG
RPX = RH * W
SPLAT_R = 2
_OFFS = [(dx, dy) for dx in range(-SPLAT_R, SPLAT_R + 1)
         for dy in range(-SPLAT_R, SPLAT_R + 1)]
_F32MAX = 3.4028235e38


def _sc_render(gauss, params, t, nv, ng):
    nchunk = ng // 16
    mesh = plsc.VectorSubcoreMesh(core_axis_name="c", subcore_axis_name="s")

    out_type = (
        jax.ShapeDtypeStruct((3 * nv * 3 * RG, RPX), jnp.float32),
        jax.ShapeDtypeStruct((3 * nv * RG, RPX), jnp.float32),
        jax.ShapeDtypeStruct((nv * RG, 128), jnp.float32),
    )
    scratch = (
        [pltpu.VMEM((11 * ng,), jnp.float32),
         pltpu.VMEM((256,), jnp.float32)]
        + [pltpu.VMEM((RPX,), jnp.float32) for _ in range(12)]
        + [pltpu.VMEM((128,), jnp.float32),
           pltpu.VMEM((ng + 16,), jnp.int32),
           pltpu.SemaphoreType.DMA]
    )

    @functools.partial(
        pl.kernel, mesh=mesh, out_type=out_type, scratch_types=scratch,
        compiler_params=pltpu.CompilerParams(needs_layout_passes=False),
    )
    def render(gauss_hbm, params_hbm, zeros_hbm, rgb_hbm, a_hbm, stat_hbm,
               gv, pvm, a0, r0, g0, b0, a1, r1, g1, b1, a2, r2, g2, b2, stv,
               idxbuf, zsem):
        wid = lax.axis_index("s") * 2 + lax.axis_index("c")
        acc_a = [a0, a1, a2]
        acc_rgb = [[r0, g0, b0], [r1, g1, b1], [r2, g2, b2]]
        accs = [a0, r0, g0, b0, a1, r1, g1, b1, a2, r2, g2, b2]
        zv = jnp.zeros((16,), jnp.float32)

        for k in range(2):
            e = wid + 32 * k
            u = e // RG
            rgn = e % RG
            ti = u // 2
            ybase = rgn * RH

            zcopies = [pltpu.async_copy(zeros_hbm, ar, zsem) for ar in accs]
            pltpu.sync_copy(gauss_hbm.at[ti], gv)
            pltpu.sync_copy(params_hbm.at[u], pvm)

            p = [pvm[pl.ds(j * 16, 16)] for j in range(16)]
            m00, m01, m02, m03, m10, m11, m12, m13, m20, m21, m22, m23 = p[:12]
            fxv, fyv, cxv, cyv = p[12], p[13], p[14], p[15]

            iota = lax.iota(jnp.int32, 16)

            def bf16r(vv):
                bits = plsc.bitcast(vv, jnp.int32)
                r = bits + 0x7FFF + ((bits >> 16) & 1)
                return plsc.bitcast(r & (-65536), jnp.float32)

            def project(x, y, z, sx, sy, sz, opc):
                camx = m00 * x + m01 * y + m02 * z + m03
                camy = m10 * x + m11 * y + m12 * z + m13
                camz = m20 * x + m21 * y + m22 * z + m23
                fin = ((jnp.abs(camx) <= _F32MAX) & (jnp.abs(camy) <= _F32MAX)
                       & (jnp.abs(camz) <= _F32MAX))
                geo = (camz > 1e-3) & fin
                v_all = geo & (opc > 1e-5)
                zs = jnp.where(v_all, camz, 1.0)
                upx = (camx * fxv) / zs + cxv
                vpx = (camy * fyv) / zs + cyv
                smean = (sx + sy + sz) / 3.0
                sig = jnp.clip(
                    (fxv + fyv) * 0.5 * jnp.abs(smean) / jnp.maximum(zs, 1e-3),
                    0.75, 10.0)
                inimg = ((upx >= -3.0) & (upx <= W + 2.0)
                         & (vpx >= -3.0) & (vpx <= H + 2.0))
                base = geo & inimg
                y0i = vpx.astype(jnp.int32)
                y0i = y0i - jnp.where(y0i.astype(jnp.float32) > vpx, 1, 0)
                return upx, vpx, sig, base, y0i

            def rbody(i, carry):
                curv, sgs, vcs = carry
                def ch(c):
                    return gv[pl.ds(c * ng + i * 16, 16)]
                x, y, z = bf16r(ch(0)), bf16r(ch(1)), bf16r(ch(2))
                opc = jnp.clip(ch(9), 0.0, 1.0)
                upx, vpx, sig, base, y0i = project(
                    x, y, z, ch(3), ch(4), ch(5), opc)
                gall = base & (opc > 1e-5)
                touch = (gall & (y0i >= ybase - SPLAT_R)
                         & (y0i <= ybase + (RH + SPLAT_R - 1)))
                tm = jnp.where(touch, 1, 0)
                pos = curv + jnp.cumsum(tm) - 1
                plsc.store_scatter(idxbuf, [pos], i * 16 + iota, mask=touch)
                curv = curv + plsc.all_reduce_population_count(touch)
                sgs = sgs + jnp.where(gall, sig, 0.0)
                vcs = vcs + jnp.where(gall, 1.0, 0.0)
                return curv, sgs, vcs

            curv, sgs, vcs = lax.fori_loop(
                0, nchunk, rbody, (jnp.zeros((16,), jnp.int32), zv, zv))
            cur = jnp.max(curv)
            sig_sum = jnp.sum(sgs)
            vcnt = jnp.sum(vcs)
            for cp in zcopies:
                cp.wait()

            def sbody(j, _):
                lm = (j * 16 + iota) < cur
                gidx = jnp.where(lm, idxbuf[pl.ds(j * 16, 16)], 0)
                def gch(c):
                    return plsc.load_gather(gv, [c * ng + gidx])
                x, y, z = bf16r(gch(0)), bf16r(gch(1)), bf16r(gch(2))
                cr = jnp.clip(gch(6), 0.0, 1.0)
                cg = jnp.clip(gch(7), 0.0, 1.0)
                cb = jnp.clip(gch(8), 0.0, 1.0)
                opc = jnp.clip(gch(9), 0.0, 1.0)
                dyn = jnp.clip(1.0 - gch(10), 0.0, 1.0)
                upx, vpx, sig, base, y0i = project(
                    x, y, z, gch(3), gch(4), gch(5), opc)
                a_all = opc
                a_dyn = opc * dyn
                a_sta = opc * (1.0 - dyn)
                gate = [lm & base & (a > 1e-5) for a in (a_all, a_dyn, a_sta)]
                alph = [a_all, a_dyn, a_sta]
                x0i = upx.astype(jnp.int32)
                x0i = x0i - jnp.where(x0i.astype(jnp.float32) > upx, 1, 0)
                ux = upx - x0i.astype(jnp.float32)
                uy = vpx - y0i.astype(jnp.float32)
                inv_s = 1.0 / sig
                yrel0 = y0i - ybase
                for dx, dy in _OFFS:
                    xo = x0i + dx
                    yrel = yrel0 + dy
                    inb = (xo >= 0) & (xo < W) & (yrel >= 0) & (yrel < RH)
                    idx = jnp.where(inb, yrel * W + xo, 0)
                    du = (ux - dx) * inv_s
                    dv = (uy - dy) * inv_s
                    wexp = jnp.exp(-0.5 * (du * du + dv * dv))
                    for bi in range(3):
                        m = inb & gate[bi]
                        w = wexp * alph[bi]
                        plsc.addupdate_scatter(acc_a[bi], [idx], w, mask=m)
                        plsc.addupdate_scatter(acc_rgb[bi][0], [idx], w * cr, mask=m)
                        plsc.addupdate_scatter(acc_rgb[bi][1], [idx], w * cg, mask=m)
                        plsc.addupdate_scatter(acc_rgb[bi][2], [idx], w * cb, mask=m)
                return 0

            lax.fori_loop(0, (cur + 15) // 16, sbody, 0)

            def nbody(i, tc):
                sl = pl.ds(i * 16, 16)
                for bi in range(3):
                    den = acc_a[bi][sl]
                    rcp = 1.0 / jnp.maximum(den, 1e-6)
                    for c in range(3):
                        acc_rgb[bi][c][sl] = acc_rgb[bi][c][sl] * rcp
                    acc_a[bi][sl] = jnp.clip(den, 0.0, 1.0)
                    if bi == 0:
                        tc = tc + jnp.where(den > 1e-6, 1.0, 0.0)
                return tc
            tc = lax.fori_loop(0, RPX // 16, nbody, zv)
            touch_cnt = jnp.sum(tc)

            for bi in range(3):
                for c in range(3):
                    row = ((bi * nv + u) * 3 + c) * RG + rgn
                    pltpu.sync_copy(acc_rgb[bi][c], rgb_hbm.at[row])
                pltpu.sync_copy(acc_a[bi], a_hbm.at[(bi * nv + u) * RG + rgn])

            io = lax.iota(jnp.int32, 16)
            st = (jnp.where(io == 0, touch_cnt, 0.0)
                  + jnp.where(io == 1, sig_sum, 0.0)
                  + jnp.where(io == 2, vcnt, 0.0))
            stv[pl.ds(0, 16)] = st
            for j in range(1, 8):
                stv[pl.ds(j * 16, 16)] = zv
            pltpu.sync_copy(stv, stat_hbm.at[e])

    return render(gauss, params, jnp.zeros((RPX,), jnp.float32))


def kernel(center, scale, feat_dc, opacity, background_prob, sem_proj_2d,
           camera_intrinsics, camera_to_world, first_ego_pose_world):
    b, t, v = center.shape[0], center.shape[1], center.shape[2]
    ng = v * center.shape[3] * center.shape[4]
    nv = t * v

    f32 = jnp.float32
    cf = center.astype(f32).reshape(t, ng, 3)
    sf = scale.astype(f32).reshape(t, ng, 3)
    col = feat_dc.astype(f32).reshape(t, ng, 3)
    op = opacity.astype(f32).reshape(t, ng)
    bg = background_prob.astype(f32).reshape(t, ng)
    gauss = jnp.stack(
        [cf[..., 0], cf[..., 1], cf[..., 2],
         sf[..., 0], sf[..., 1], sf[..., 2],
         col[..., 0], col[..., 1], col[..., 2], op, bg], axis=1)
    gauss = gauss.reshape(t, 11 * ng)

    fp = first_ego_pose_world[0].astype(f32)
    w2c = jnp.stack([jnp.linalg.inv(camera_to_world[0, ti].astype(f32))
                     for ti in range(t)])
    m = (w2c @ fp)[..., :3, :]
    m = m.astype(jnp.bfloat16).astype(f32)
    mflat = m.reshape(nv, 12)
    intr = jnp.tile(camera_intrinsics[0].astype(f32), (t, 1))
    pv = jnp.concatenate([mflat, intr], axis=1)
    params = jnp.broadcast_to(pv[:, :, None], (nv, 16, 16)).reshape(nv, 256)

    rgb_flat, a_flat, stat = _sc_render(gauss, params, t, nv, ng)

    rgb = rgb_flat.reshape(3, b, t, v, 3, H, W)
    a_img = a_flat.reshape(3, b, t, v, 1, H, W)
    touch = jnp.sum(stat[:, 0]) / (nv * H * W)
    sig_rows = stat[0::RG]
    sig = jnp.mean(sig_rows[:, 1] / jnp.maximum(sig_rows[:, 2], 1.0))
    return (rgb[2], rgb[1], rgb[0], a_img[2], a_img[1], a_img[0],
            sem_proj_2d, sig, touch)


# speedup vs baseline: 1.1612x; 1.1612x over previous
import functools

import jax
import jax.numpy as jnp
from jax import lax
from jax.experimental import pallas as pl
from jax.experimental.pallas import tpu as pltpu
from jax.experimental.pallas import tpu_sc as plsc

H, W = 224, 448
RG = 16
RH = H // RG
RPX = RH * W
SPLAT_R = 2
_OFFS = [(dx, dy) for dx in range(-SPLAT_R, SPLAT_R + 1)
         for dy in range(-SPLAT_R, SPLAT_R + 1)]
_F32MAX = 3.4028235e38


def _sc_render(gauss, params, t, nv, ng):
    nchunk = ng // 16
    mesh = plsc.VectorSubcoreMesh(core_axis_name="c", subcore_axis_name="s")

    out_type = (
        jax.ShapeDtypeStruct((3 * nv * 3 * RG, RPX), jnp.float32),
        jax.ShapeDtypeStruct((3 * nv * RG, RPX), jnp.float32),
        jax.ShapeDtypeStruct((nv * RG, 128), jnp.float32),
    )
    scratch = (
        [pltpu.VMEM((11 * ng,), jnp.float32),
         pltpu.VMEM((256,), jnp.float32)]
        + [pltpu.VMEM((RPX,), jnp.float32) for _ in range(12)]
        + [pltpu.VMEM((128,), jnp.float32),
           pltpu.VMEM((ng + 16,), jnp.int32)]
    )

    @functools.partial(
        pl.kernel, mesh=mesh, out_type=out_type, scratch_types=scratch,
        compiler_params=pltpu.CompilerParams(needs_layout_passes=False),
    )
    def render(gauss_hbm, params_hbm, rgb_hbm, a_hbm, stat_hbm,
               gv, pvm, a0, r0, g0, b0, a1, r1, g1, b1, a2, r2, g2, b2, stv,
               idxbuf):
        wid = lax.axis_index("s") * 2 + lax.axis_index("c")
        acc_a = [a0, a1, a2]
        acc_rgb = [[r0, g0, b0], [r1, g1, b1], [r2, g2, b2]]
        accs = [a0, r0, g0, b0, a1, r1, g1, b1, a2, r2, g2, b2]
        zv = jnp.zeros((16,), jnp.float32)

        for k in range(2):
            e = wid + 32 * k
            u = e // RG
            rgn = e % RG
            ti = u // 2
            ybase = rgn * RH

            pltpu.sync_copy(gauss_hbm.at[ti], gv)
            pltpu.sync_copy(params_hbm.at[u], pvm)

            def zbody(i, _):
                for ar in accs:
                    ar[pl.ds(i * 16, 16)] = zv
                return 0
            lax.fori_loop(0, RPX // 16, zbody, 0)

            p = [pvm[pl.ds(j * 16, 16)] for j in range(16)]
            m00, m01, m02, m03, m10, m11, m12, m13, m20, m21, m22, m23 = p[:12]
            fxv, fyv, cxv, cyv = p[12], p[13], p[14], p[15]

            iota = lax.iota(jnp.int32, 16)

            def bf16r(vv):
                bits = plsc.bitcast(vv, jnp.int32)
                r = bits + 0x7FFF + ((bits >> 16) & 1)
                return plsc.bitcast(r & (-65536), jnp.float32)

            def project(x, y, z, sx, sy, sz, opc):
                camx = m00 * x + m01 * y + m02 * z + m03
                camy = m10 * x + m11 * y + m12 * z + m13
                camz = m20 * x + m21 * y + m22 * z + m23
                fin = ((jnp.abs(camx) <= _F32MAX) & (jnp.abs(camy) <= _F32MAX)
                       & (jnp.abs(camz) <= _F32MAX))
                geo = (camz > 1e-3) & fin
                v_all = geo & (opc > 1e-5)
                zs = jnp.where(v_all, camz, 1.0)
                upx = (camx * fxv) / zs + cxv
                vpx = (camy * fyv) / zs + cyv
                smean = (sx + sy + sz) / 3.0
                sig = jnp.clip(
                    (fxv + fyv) * 0.5 * jnp.abs(smean) / jnp.maximum(zs, 1e-3),
                    0.75, 10.0)
                inimg = ((upx >= -3.0) & (upx <= W + 2.0)
                         & (vpx >= -3.0) & (vpx <= H + 2.0))
                base = geo & inimg
                y0i = vpx.astype(jnp.int32)
                y0i = y0i - jnp.where(y0i.astype(jnp.float32) > vpx, 1, 0)
                return upx, vpx, sig, base, y0i

            def rbody(i, carry):
                curv, sgs, vcs = carry
                def ch(c):
                    return gv[pl.ds(c * ng + i * 16, 16)]
                x, y, z = bf16r(ch(0)), bf16r(ch(1)), bf16r(ch(2))
                opc = jnp.clip(ch(9), 0.0, 1.0)
                upx, vpx, sig, base, y0i = project(
                    x, y, z, ch(3), ch(4), ch(5), opc)
                gall = base & (opc > 1e-5)
                touch = (gall & (y0i >= ybase - SPLAT_R)
                         & (y0i <= ybase + (RH + SPLAT_R - 1)))
                tm = jnp.where(touch, 1, 0)
                pos = curv + jnp.cumsum(tm) - 1
                plsc.store_scatter(idxbuf, [pos], i * 16 + iota, mask=touch)
                curv = curv + plsc.all_reduce_population_count(touch)
                sgs = sgs + jnp.where(gall, sig, 0.0)
                vcs = vcs + jnp.where(gall, 1.0, 0.0)
                return curv, sgs, vcs

            curv, sgs, vcs = lax.fori_loop(
                0, nchunk, rbody, (jnp.zeros((16,), jnp.int32), zv, zv))
            cur = jnp.max(curv)
            sig_sum = jnp.sum(sgs)
            vcnt = jnp.sum(vcs)

            def sbody(j, _):
                lm = (j * 16 + iota) < cur
                gidx = jnp.where(lm, idxbuf[pl.ds(j * 16, 16)], 0)
                def gch(c):
                    return plsc.load_gather(gv, [c * ng + gidx])
                x, y, z = bf16r(gch(0)), bf16r(gch(1)), bf16r(gch(2))
                cr = jnp.clip(gch(6), 0.0, 1.0)
                cg = jnp.clip(gch(7), 0.0, 1.0)
                cb = jnp.clip(gch(8), 0.0, 1.0)
                opc = jnp.clip(gch(9), 0.0, 1.0)
                dyn = jnp.clip(1.0 - gch(10), 0.0, 1.0)
                upx, vpx, sig, base, y0i = project(
                    x, y, z, gch(3), gch(4), gch(5), opc)
                a_all = opc
                a_dyn = opc * dyn
                a_sta = opc * (1.0 - dyn)
                gate = [lm & base & (a > 1e-5) for a in (a_all, a_dyn, a_sta)]
                alph = [a_all, a_dyn, a_sta]
                x0i = upx.astype(jnp.int32)
                x0i = x0i - jnp.where(x0i.astype(jnp.float32) > upx, 1, 0)
                ux = upx - x0i.astype(jnp.float32)
                uy = vpx - y0i.astype(jnp.float32)
                inv_s = 1.0 / sig
                yrel0 = y0i - ybase
                for dx, dy in _OFFS:
                    xo = x0i + dx
                    yrel = yrel0 + dy
                    inb = (xo >= 0) & (xo < W) & (yrel >= 0) & (yrel < RH)
                    idx = jnp.where(inb, yrel * W + xo, 0)
                    du = (ux - dx) * inv_s
                    dv = (uy - dy) * inv_s
                    wexp = jnp.exp(-0.5 * (du * du + dv * dv))
                    for bi in range(3):
                        m = inb & gate[bi]
                        w = wexp * alph[bi]
                        plsc.addupdate_scatter(acc_a[bi], [idx], w, mask=m)
                        plsc.addupdate_scatter(acc_rgb[bi][0], [idx], w * cr, mask=m)
                        plsc.addupdate_scatter(acc_rgb[bi][1], [idx], w * cg, mask=m)
                        plsc.addupdate_scatter(acc_rgb[bi][2], [idx], w * cb, mask=m)
                return 0

            lax.fori_loop(0, (cur + 15) // 16, sbody, 0)

            def nbody(i, tc):
                sl = pl.ds(i * 16, 16)
                for bi in range(3):
                    den = acc_a[bi][sl]
                    rcp = 1.0 / jnp.maximum(den, 1e-6)
                    for c in range(3):
                        acc_rgb[bi][c][sl] = acc_rgb[bi][c][sl] * rcp
                    acc_a[bi][sl] = jnp.clip(den, 0.0, 1.0)
                    if bi == 0:
                        tc = tc + jnp.where(den > 1e-6, 1.0, 0.0)
                return tc
            tc = lax.fori_loop(0, RPX // 16, nbody, zv)
            touch_cnt = jnp.sum(tc)

            for bi in range(3):
                for c in range(3):
                    row = ((bi * nv + u) * 3 + c) * RG + rgn
                    pltpu.sync_copy(acc_rgb[bi][c], rgb_hbm.at[row])
                pltpu.sync_copy(acc_a[bi], a_hbm.at[(bi * nv + u) * RG + rgn])

            io = lax.iota(jnp.int32, 16)
            st = (jnp.where(io == 0, touch_cnt, 0.0)
                  + jnp.where(io == 1, sig_sum, 0.0)
                  + jnp.where(io == 2, vcnt, 0.0))
            stv[pl.ds(0, 16)] = st
            for j in range(1, 8):
                stv[pl.ds(j * 16, 16)] = zv
            pltpu.sync_copy(stv, stat_hbm.at[e])

    return render(gauss, params)


def kernel(center, scale, feat_dc, opacity, background_prob, sem_proj_2d,
           camera_intrinsics, camera_to_world, first_ego_pose_world):
    b, t, v = center.shape[0], center.shape[1], center.shape[2]
    ng = v * center.shape[3] * center.shape[4]
    nv = t * v

    f32 = jnp.float32
    cf = center.astype(f32).reshape(t, ng, 3)
    sf = scale.astype(f32).reshape(t, ng, 3)
    col = feat_dc.astype(f32).reshape(t, ng, 3)
    op = opacity.astype(f32).reshape(t, ng)
    bg = background_prob.astype(f32).reshape(t, ng)
    gauss = jnp.stack(
        [cf[..., 0], cf[..., 1], cf[..., 2],
         sf[..., 0], sf[..., 1], sf[..., 2],
         col[..., 0], col[..., 1], col[..., 2], op, bg], axis=1)
    gauss = gauss.reshape(t, 11 * ng)

    fp = first_ego_pose_world[0].astype(f32)
    w2c = jnp.stack([jnp.linalg.inv(camera_to_world[0, ti].astype(f32))
                     for ti in range(t)])
    m = (w2c @ fp)[..., :3, :]
    m = m.astype(jnp.bfloat16).astype(f32)
    mflat = m.reshape(nv, 12)
    intr = jnp.tile(camera_intrinsics[0].astype(f32), (t, 1))
    pv = jnp.concatenate([mflat, intr], axis=1)
    params = jnp.broadcast_to(pv[:, :, None], (nv, 16, 16)).reshape(nv, 256)

    rgb_flat, a_flat, stat = _sc_render(gauss, params, t, nv, ng)

    rgb = rgb_flat.reshape(3, b, t, v, 3, H, W)
    a_img = a_flat.reshape(3, b, t, v, 1, H, W)
    touch = jnp.sum(stat[:, 0]) / (nv * H * W)
    sig_rows = stat[0::RG]
    sig = jnp.mean(sig_rows[:, 1] / jnp.maximum(sig_rows[:, 2], 1.0))
    return (rgb[2], rgb[1], rgb[0], a_img[2], a_img[1], a_img[0],
            sem_proj_2d, sig, touch)
